# Initial kernel scaffold; baseline (speedup 1.0000x reference)
import jax
import jax.numpy as jnp
from jax.experimental import pallas as pl


def kernel(x, batch, degree, W1, b1, W2, b2, Wh, bh, Wa, ba):
    raise NotImplementedError("write your pallas kernel here")


# speedup vs baseline: 11.0151x; 11.0151x over previous
import functools

import jax
import jax.numpy as jnp
from jax import lax
from jax.experimental import pallas as pl
from jax.experimental.pallas import tpu as pltpu
from jax.experimental.pallas import tpu_sc as plsc

N = 100000
D = 128
H = 64
G = 1000

NTILES = 16
CHUNK = 6272
NP = NTILES * CHUNK
NVEC = CHUNK // 16
GP = 1024
ROWS = GP // 16

MLP_BLK = 2048
MLP_GRID = NP // MLP_BLK


def _mlp_body(x_ref, w1_ref, b1_ref, w2_ref, b2_ref, whaT_ref, bhaT_ref,
              ha_ref):
    xb = x_ref[...]
    h1 = jnp.maximum(
        jnp.dot(xb, w1_ref[...], preferred_element_type=jnp.float32)
        + b1_ref[...], 0.0)
    h2 = jnp.maximum(
        jnp.dot(h1, w2_ref[...], preferred_element_type=jnp.float32)
        + b2_ref[...], 0.0)
    haT = lax.dot_general(
        whaT_ref[...], h2, (((1,), (1,)), ((), ())),
        preferred_element_type=jnp.float32) + bhaT_ref[...]
    ha_ref[...] = haT


def _run_mlp(x, W1, b1, W2, b2, WhaT, bhaT):
    full = lambda s: pl.BlockSpec(s, lambda i: (0, 0))
    return pl.pallas_call(
        _mlp_body,
        grid=(MLP_GRID,),
        in_specs=[
            pl.BlockSpec((MLP_BLK, D), lambda i: (i, 0)),
            full((D, H)),
            full((1, H)),
            full((H, H)),
            full((1, H)),
            full((2, H)),
            full((2, 1)),
        ],
        out_specs=pl.BlockSpec((2, MLP_BLK), lambda i: (0, i)),
        out_shape=jax.ShapeDtypeStruct((2, NP), jnp.float32),
    )(x, W1, b1, W2, b2, WhaT, bhaT)


def _combine(sRow, sComb, tmp, slice128, slot):
    for r in range(NTILES):
        pltpu.sync_copy(sRow.at[pl.ds(GP * r + 128 * slot, 128)],
                        tmp.at[pl.ds(128 * r, 128)])
    for v in range(8):
        acc = tmp[pl.ds(16 * v, 16)]
        for r in range(1, NTILES):
            acc = acc + tmp[pl.ds(128 * r + 16 * v, 16)]
        slice128[pl.ds(16 * v, 16)] = acc
    pltpu.sync_copy(slice128, sComb.at[pl.ds(128 * slot, 128)])


def _seg_body(h_hbm, aug_hbm, deg_hbm, b_hbm, out_hbm,
              hb, eb, augb, degb, bb, deltab,
              E, Hs, C, Dl, Ef, Hf, Cf, Df,
              outb, tmp, slice128,
              sE, sH, sC, sD, sEc, sHc, sCc, sDc):
    wid = lax.axis_index("s")
    base = wid * CHUNK

    zeros16 = jnp.zeros((16,), jnp.float32)
    ones16 = jnp.full((16,), 1.0, jnp.float32)
    iota16 = lax.iota(jnp.int32, 16)

    pltpu.sync_copy(h_hbm.at[pl.ds(base, CHUNK)], hb)
    pltpu.sync_copy(aug_hbm.at[pl.ds(base, CHUNK)], augb)
    pltpu.sync_copy(deg_hbm.at[pl.ds(base, CHUNK)], degb)
    pltpu.sync_copy(b_hbm.at[pl.ds(base, CHUNK)], bb)

    def zloop(r, _):
        s = r * 16
        E[pl.ds(s, 16)] = zeros16
        Hs[pl.ds(s, 16)] = zeros16
        C[pl.ds(s, 16)] = zeros16
        Dl[pl.ds(s, 16)] = zeros16
        return 0
    lax.fori_loop(0, ROWS, zloop, 0)

    def passa(i, _):
        s = i * 16
        b = bb[pl.ds(s, 16)]
        hv = hb[pl.ds(s, 16)]
        ev = jnp.exp(hv / 5.0)
        eb[pl.ds(s, 16)] = ev
        plsc.addupdate_scatter(E, [b], ev)
        plsc.addupdate_scatter(Hs, [b], hv)
        plsc.addupdate_scatter(C, [b], ones16)
        return 0
    lax.fori_loop(0, NVEC, passa, 0)

    pltpu.sync_copy(E, sE.at[pl.ds(GP * wid, GP)])
    pltpu.sync_copy(Hs, sH.at[pl.ds(GP * wid, GP)])
    pltpu.sync_copy(C, sC.at[pl.ds(GP * wid, GP)])
    plsc.subcore_barrier()

    @pl.when(wid < 8)
    def _():
        _combine(sE, sEc, tmp, slice128, wid)
        _combine(sC, sCc, tmp, slice128, wid)

    @pl.when(wid >= 8)
    def _():
        _combine(sH, sHc, tmp, slice128, wid - 8)

    plsc.subcore_barrier()
    pltpu.sync_copy(sEc, Ef)
    pltpu.sync_copy(sHc, Hf)
    pltpu.sync_copy(sCc, Cf)

    def passb(i, _):
        s = i * 16
        b = bb[pl.ds(s, 16)]
        hp = plsc.load_gather(Hf, [b])
        delta = jnp.abs(hp - augb[pl.ds(s, 16)]) / (degb[pl.ds(s, 16)] + 1.0)
        deltab[pl.ds(s, 16)] = delta
        plsc.addupdate_scatter(Dl, [b], delta)
        return 0
    lax.fori_loop(0, NVEC, passb, 0)

    pltpu.sync_copy(Dl, sD.at[pl.ds(GP * wid, GP)])
    plsc.subcore_barrier()

    @pl.when(wid < 8)
    def _():
        _combine(sD, sDc, tmp, slice128, wid)

    plsc.subcore_barrier()
    pltpu.sync_copy(sDc, Df)

    zi16 = jnp.zeros((16,), jnp.int32)
    oi16 = jnp.full((16,), 1, jnp.int32)
    def passc(i, _):
        s = i * 16
        b = bb[pl.ds(s, 16)]
        Ev = plsc.load_gather(Ef, [b])
        Cv = plsc.load_gather(Cf, [b])
        Dv = plsc.load_gather(Df, [b])
        Lv = eb[pl.ds(s, 16)] / Ev
        avg = Dv / jnp.maximum(Cv, 1.0)
        delta = deltab[pl.ds(s, 16)]
        bl = jnp.where(delta >= avg, 1.0, 0.0)
        rows = iota16 + s
        plsc.store_scatter(outb, [rows, zi16], Lv)
        plsc.store_scatter(outb, [rows, oi16], bl)
        return 0
    lax.fori_loop(0, NVEC, passc, 0)

    pltpu.sync_copy(outb, out_hbm.at[pl.ds(base, CHUNK), :])


def _run_seg(h, aug, deg, b32):
    mesh = plsc.VectorSubcoreMesh(
        core_axis_name="c", subcore_axis_name="s", num_cores=1)
    f32 = jnp.float32
    kern = functools.partial(
        pl.kernel,
        out_type=jax.ShapeDtypeStruct((NP, 2), f32),
        mesh=mesh,
        compiler_params=pltpu.CompilerParams(
            needs_layout_passes=False, use_tc_tiling_on_sc=False),
        scratch_types=[
            pltpu.VMEM((CHUNK,), f32),
            pltpu.VMEM((CHUNK,), f32),
            pltpu.VMEM((CHUNK,), f32),
            pltpu.VMEM((CHUNK,), f32),
            pltpu.VMEM((CHUNK,), jnp.int32),
            pltpu.VMEM((CHUNK,), f32),
            pltpu.VMEM((GP,), f32),
            pltpu.VMEM((GP,), f32),
            pltpu.VMEM((GP,), f32),
            pltpu.VMEM((GP,), f32),
            pltpu.VMEM((GP,), f32),
            pltpu.VMEM((GP,), f32),
            pltpu.VMEM((GP,), f32),
            pltpu.VMEM((GP,), f32),
            pltpu.VMEM((CHUNK, 2), f32),
            pltpu.VMEM((NTILES * 128,), f32),
            pltpu.VMEM((128,), f32),
            pltpu.VMEM_SHARED((NTILES * GP,), f32),
            pltpu.VMEM_SHARED((NTILES * GP,), f32),
            pltpu.VMEM_SHARED((NTILES * GP,), f32),
            pltpu.VMEM_SHARED((NTILES * GP,), f32),
            pltpu.VMEM_SHARED((GP,), f32),
            pltpu.VMEM_SHARED((GP,), f32),
            pltpu.VMEM_SHARED((GP,), f32),
            pltpu.VMEM_SHARED((GP,), f32),
        ],
    )(_seg_body)
    return kern(h, aug, deg, b32)


def kernel(x, batch, degree, W1, b1, W2, b2, Wh, bh, Wa, ba):
    WhaT = jnp.concatenate([Wh, Wa], axis=1).T
    bhaT = jnp.concatenate([bh, ba]).reshape(2, 1)
    ha = _run_mlp(x, W1, b1.reshape(1, H), W2, b2.reshape(1, H),
                  WhaT, bhaT)
    pad = NP - N
    b32 = jnp.pad(batch.astype(jnp.int32), (0, pad), constant_values=G)
    degp = jnp.pad(degree, (0, pad))
    out = _run_seg(ha[0], ha[1], degp, b32)
    return out[:N]


# speedup vs baseline: 18.9519x; 1.7205x over previous
import functools

import jax
import jax.numpy as jnp
from jax import lax
from jax.experimental import pallas as pl
from jax.experimental.pallas import tpu as pltpu
from jax.experimental.pallas import tpu_sc as plsc

N = 100000
D = 128
H = 64
G = 1000

NTILES = 16
CHUNK = 6272
NP = NTILES * CHUNK
NVEC = CHUNK // 16
GP = 1024
ROWS = GP // 16

MLP_BLK = 2048
MLP_GRID = NP // MLP_BLK


def _mlp_body(x_ref, w1_ref, b1_ref, w2_ref, b2_ref, whaT_ref, bhaT_ref,
              ha_ref):
    xb = x_ref[...]
    h1 = jnp.maximum(
        jnp.dot(xb, w1_ref[...], preferred_element_type=jnp.float32)
        + b1_ref[...], 0.0)
    h2 = jnp.maximum(
        jnp.dot(h1, w2_ref[...], preferred_element_type=jnp.float32)
        + b2_ref[...], 0.0)
    haT = lax.dot_general(
        whaT_ref[...], h2, (((1,), (1,)), ((), ())),
        preferred_element_type=jnp.float32) + bhaT_ref[...]
    ha_ref[...] = haT


def _run_mlp(x, W1, b1, W2, b2, WhaT, bhaT):
    full = lambda s: pl.BlockSpec(s, lambda i: (0, 0))
    return pl.pallas_call(
        _mlp_body,
        grid=(MLP_GRID,),
        in_specs=[
            pl.BlockSpec((MLP_BLK, D), lambda i: (i, 0)),
            full((D, H)),
            full((1, H)),
            full((H, H)),
            full((1, H)),
            full((2, H)),
            full((2, 1)),
        ],
        out_specs=pl.BlockSpec((2, MLP_BLK), lambda i: (0, i)),
        out_shape=jax.ShapeDtypeStruct((2, NP), jnp.float32),
    )(x, W1, b1, W2, b2, WhaT, bhaT)


def _combine(sRow, sComb, tmp, slice128, slot):
    for r in range(NTILES):
        pltpu.sync_copy(sRow.at[pl.ds(GP * r + 128 * slot, 128)],
                        tmp.at[pl.ds(128 * r, 128)])
    for v in range(8):
        acc = tmp[pl.ds(16 * v, 16)]
        for r in range(1, NTILES):
            acc = acc + tmp[pl.ds(128 * r + 16 * v, 16)]
        slice128[pl.ds(16 * v, 16)] = acc
    pltpu.sync_copy(slice128, sComb.at[pl.ds(128 * slot, 128)])


def _seg_body(h_hbm, aug_hbm, deg_hbm, b_hbm, out_hbm,
              hb, eb, augb, degb, bb, deltab,
              E, Hs, C, Dl, Ef, Hf, Cf, Df,
              Lb, Bb, tmp, slice128,
              sE, sH, sC, sD, sEc, sHc, sCc, sDc):
    wid = lax.axis_index("s")
    base = wid * CHUNK

    zeros16 = jnp.zeros((16,), jnp.float32)
    ones16 = jnp.full((16,), 1.0, jnp.float32)
    iota16 = lax.iota(jnp.int32, 16)

    pltpu.sync_copy(h_hbm.at[pl.ds(base, CHUNK)], hb)
    pltpu.sync_copy(aug_hbm.at[pl.ds(base, CHUNK)], augb)
    pltpu.sync_copy(deg_hbm.at[pl.ds(base, CHUNK)], degb)
    pltpu.sync_copy(b_hbm.at[pl.ds(base, CHUNK)], bb)

    def zloop(r, _):
        s = r * 16
        E[pl.ds(s, 16)] = zeros16
        Hs[pl.ds(s, 16)] = zeros16
        C[pl.ds(s, 16)] = zeros16
        Dl[pl.ds(s, 16)] = zeros16
        return 0
    lax.fori_loop(0, ROWS, zloop, 0)

    def passa(i, _):
        s = i * 16
        b = bb[pl.ds(s, 16)]
        hv = hb[pl.ds(s, 16)]
        ev = jnp.exp(hv / 5.0)
        eb[pl.ds(s, 16)] = ev
        plsc.addupdate_scatter(E, [b], ev)
        plsc.addupdate_scatter(Hs, [b], hv)
        plsc.addupdate_scatter(C, [b], ones16)
        return 0
    lax.fori_loop(0, NVEC, passa, 0)

    pltpu.sync_copy(E, sE.at[pl.ds(GP * wid, GP)])
    pltpu.sync_copy(Hs, sH.at[pl.ds(GP * wid, GP)])
    pltpu.sync_copy(C, sC.at[pl.ds(GP * wid, GP)])
    plsc.subcore_barrier()

    @pl.when(wid < 8)
    def _():
        _combine(sE, sEc, tmp, slice128, wid)
        _combine(sC, sCc, tmp, slice128, wid)

    @pl.when(wid >= 8)
    def _():
        _combine(sH, sHc, tmp, slice128, wid - 8)

    plsc.subcore_barrier()
    pltpu.sync_copy(sEc, Ef)
    pltpu.sync_copy(sHc, Hf)
    pltpu.sync_copy(sCc, Cf)

    def passb(i, _):
        s = i * 16
        b = bb[pl.ds(s, 16)]
        hp = plsc.load_gather(Hf, [b])
        delta = jnp.abs(hp - augb[pl.ds(s, 16)]) / (degb[pl.ds(s, 16)] + 1.0)
        deltab[pl.ds(s, 16)] = delta
        plsc.addupdate_scatter(Dl, [b], delta)
        return 0
    lax.fori_loop(0, NVEC, passb, 0)

    pltpu.sync_copy(Dl, sD.at[pl.ds(GP * wid, GP)])
    plsc.subcore_barrier()

    @pl.when(wid < 8)
    def _():
        _combine(sD, sDc, tmp, slice128, wid)

    plsc.subcore_barrier()
    pltpu.sync_copy(sDc, Df)

    def passc(i, _):
        s = i * 16
        b = bb[pl.ds(s, 16)]
        Ev = plsc.load_gather(Ef, [b])
        Cv = plsc.load_gather(Cf, [b])
        Dv = plsc.load_gather(Df, [b])
        Lv = eb[pl.ds(s, 16)] / Ev
        avg = Dv / jnp.maximum(Cv, 1.0)
        delta = deltab[pl.ds(s, 16)]
        Lb[pl.ds(s, 16)] = Lv
        Bb[pl.ds(s, 16)] = jnp.where(delta >= avg, 1.0, 0.0)
        return 0
    lax.fori_loop(0, NVEC, passc, 0)

    pltpu.sync_copy(Lb, out_hbm.at[0, pl.ds(base, CHUNK)])
    pltpu.sync_copy(Bb, out_hbm.at[1, pl.ds(base, CHUNK)])


def _run_seg(h, aug, deg, b32):
    mesh = plsc.VectorSubcoreMesh(
        core_axis_name="c", subcore_axis_name="s", num_cores=1)
    f32 = jnp.float32
    kern = functools.partial(
        pl.kernel,
        out_type=jax.ShapeDtypeStruct((2, NP), f32),
        mesh=mesh,
        compiler_params=pltpu.CompilerParams(
            needs_layout_passes=False, use_tc_tiling_on_sc=False),
        scratch_types=[
            pltpu.VMEM((CHUNK,), f32),
            pltpu.VMEM((CHUNK,), f32),
            pltpu.VMEM((CHUNK,), f32),
            pltpu.VMEM((CHUNK,), f32),
            pltpu.VMEM((CHUNK,), jnp.int32),
            pltpu.VMEM((CHUNK,), f32),
            pltpu.VMEM((GP,), f32),
            pltpu.VMEM((GP,), f32),
            pltpu.VMEM((GP,), f32),
            pltpu.VMEM((GP,), f32),
            pltpu.VMEM((GP,), f32),
            pltpu.VMEM((GP,), f32),
            pltpu.VMEM((GP,), f32),
            pltpu.VMEM((GP,), f32),
            pltpu.VMEM((CHUNK,), f32),
            pltpu.VMEM((CHUNK,), f32),
            pltpu.VMEM((NTILES * 128,), f32),
            pltpu.VMEM((128,), f32),
            pltpu.VMEM_SHARED((NTILES * GP,), f32),
            pltpu.VMEM_SHARED((NTILES * GP,), f32),
            pltpu.VMEM_SHARED((NTILES * GP,), f32),
            pltpu.VMEM_SHARED((NTILES * GP,), f32),
            pltpu.VMEM_SHARED((GP,), f32),
            pltpu.VMEM_SHARED((GP,), f32),
            pltpu.VMEM_SHARED((GP,), f32),
            pltpu.VMEM_SHARED((GP,), f32),
        ],
    )(_seg_body)
    return kern(h, aug, deg, b32)


def kernel(x, batch, degree, W1, b1, W2, b2, Wh, bh, Wa, ba):
    WhaT = jnp.concatenate([Wh, Wa], axis=1).T
    bhaT = jnp.concatenate([bh, ba]).reshape(2, 1)
    ha = _run_mlp(x, W1, b1.reshape(1, H), W2, b2.reshape(1, H),
                  WhaT, bhaT)
    pad = NP - N
    b32 = jnp.pad(batch.astype(jnp.int32), (0, pad), constant_values=G)
    degp = jnp.pad(degree, (0, pad))
    out = _run_seg(ha[0], ha[1], degp, b32)
    return out[:, :N].T
